# baseline (speedup 1.0000x reference)
import functools

import jax
import jax.numpy as jnp
from jax.experimental import pallas as pl

_BM = 400


def _g_kernel(x_ref, w1_ref, g_ref):
    g_ref[...] = jnp.dot(x_ref[...], w1_ref[...],
                         preferred_element_type=jnp.float32)


def _pass1_kernel(a_ref, g_ref, w2_ref, h2_ref):
    acc = jnp.dot(a_ref[...], g_ref[...],
                  preferred_element_type=jnp.float32)
    h1 = jnp.maximum(acc, 0.0)
    h2_ref[...] = jnp.dot(h1, w2_ref[...],
                          preferred_element_type=jnp.float32)


def _pass2_kernel(a_ref, h2_ref, out_ref):
    z = jnp.dot(a_ref[...], h2_ref[...],
                preferred_element_type=jnp.float32)
    m = jnp.max(z, axis=1, keepdims=True)
    zs = z - m
    lse = jnp.log(jnp.sum(jnp.exp(zs), axis=1, keepdims=True))
    out_ref[...] = zs - lse


@functools.partial(jax.jit, static_argnames=())
def kernel(x, adjacency, W1, W2):
    n, dim_in = x.shape
    dim_h = W1.shape[1]
    dim_out = W2.shape[1]
    nb = n // _BM

    g = pl.pallas_call(
        _g_kernel,
        out_shape=jax.ShapeDtypeStruct((n, dim_h), jnp.float32),
    )(x, W1)

    h2 = pl.pallas_call(
        _pass1_kernel,
        grid=(nb,),
        in_specs=[
            pl.BlockSpec((_BM, n), lambda i: (i, 0)),
            pl.BlockSpec((n, dim_h), lambda i: (0, 0)),
            pl.BlockSpec((dim_h, dim_out), lambda i: (0, 0)),
        ],
        out_specs=pl.BlockSpec((_BM, dim_out), lambda i: (i, 0)),
        out_shape=jax.ShapeDtypeStruct((n, dim_out), jnp.float32),
    )(adjacency, g, W2)

    out = pl.pallas_call(
        _pass2_kernel,
        grid=(nb,),
        in_specs=[
            pl.BlockSpec((_BM, n), lambda i: (i, 0)),
            pl.BlockSpec((n, dim_out), lambda i: (0, 0)),
        ],
        out_specs=pl.BlockSpec((_BM, dim_out), lambda i: (i, 0)),
        out_shape=jax.ShapeDtypeStruct((n, dim_out), jnp.float32),
    )(adjacency, h2)
    return out
